# baseline (speedup 1.0000x reference)
import functools

import jax
import jax.numpy as jnp
from jax import lax
from jax.experimental import pallas as pl
from jax.experimental.pallas import tpu as pltpu
from jax.experimental.pallas import tpu_sc as plsc

E = 320000
D = 128
W = D // 2
NCORES = 2
NSUB = 16
NW = NCORES * NSUB
EPW = E // NW
C = 80
NCHUNK = EPW // C
NG = C // 16
JW = 8
NBUF = 5
FULL_ROUNDS = NCHUNK // NBUF - 1
GROUP_SETS = ((0, 1), (2, 3), (4,))
assert C % 16 == 0 and C % 8 == 0 and EPW % C == 0 and W % JW == 0

_mesh = plsc.VectorSubcoreMesh(core_axis_name="c", subcore_axis_name="s")


@functools.partial(
    pl.kernel,
    out_type=jax.ShapeDtypeStruct((E,), jnp.float32),
    mesh=_mesh,
    scratch_types=(
        [pltpu.VMEM((EPW,), jnp.int32)] * 2
        + [pltpu.VMEM((C, W), jnp.int32)] * (2 * NBUF)
        + [pltpu.VMEM((EPW,), jnp.float32)]
        + [pltpu.SemaphoreType.DMA] * (2 * NBUF)
    ),
    compiler_params=pltpu.CompilerParams(
        needs_layout_passes=False, use_tc_tiling_on_sc=False),
)
def _decode(zw_hbm, src_hbm, dst_hbm, out_hbm, src_v, dst_v, *rest):
    rows = rest[:2 * NBUF]
    out_v = rest[2 * NBUF]
    sems = rest[2 * NBUF + 1:]
    bufs = tuple(
        (rows[2 * b], rows[2 * b + 1], sems[2 * b], sems[2 * b + 1])
        for b in range(NBUF))

    wid = lax.axis_index("s") * NCORES + lax.axis_index("c")
    base = wid * EPW
    pltpu.sync_copy(src_hbm.at[pl.ds(base, EPW)], src_v)
    pltpu.sync_copy(dst_hbm.at[pl.ds(base, EPW)], dst_v)

    lane = lax.iota(jnp.int32, 16)

    def fire(ci, b):
        rs, rd, ss, sd = bufs[b]
        cb = ci * C
        pltpu.async_copy(zw_hbm.at[src_v.at[pl.ds(cb, C)]], rs, ss)
        pltpu.async_copy(zw_hbm.at[dst_v.at[pl.ds(cb, C)]], rd, sd)

    def drain(ci, b):
        rs, rd, ss, sd = bufs[b]
        cb = ci * C
        pltpu.make_async_copy(
            zw_hbm.at[src_v.at[pl.ds(cb, C)]], rs, ss).wait()
        pltpu.make_async_copy(
            zw_hbm.at[dst_v.at[pl.ds(cb, C)]], rd, sd).wait()

    def compute(ci, b):
        rows_s, rows_d, _, _ = bufs[b]
        cb = ci * C

        for groups in GROUP_SETS:
            zero = jnp.zeros((16,), jnp.float32)
            ridxs = [lane + g * 16 for g in groups]

            def jstep(jc, accs, ridxs=ridxs):
                new = list(accs)
                jb = jc * JW
                for k in range(JW):
                    cw = (lane + (jb + k)) & (W - 1)
                    for gi, ridx in enumerate(ridxs):
                        ws = plsc.load_gather(rows_s, [ridx, cw])
                        wd = plsc.load_gather(rows_d, [ridx, cw])
                        prod = (plsc.bitcast(ws, jnp.bfloat16)
                                * plsc.bitcast(wd, jnp.bfloat16))
                        lo, hi = plsc.unpack(
                            prod, format=plsc.PackFormat.INTERLEAVED)
                        ai = gi * 4 + (k % 2) * 2
                        new[ai] = new[ai] + lo
                        new[ai + 1] = new[ai + 1] + hi
                return tuple(new)

            accs = plsc.parallel_loop(
                0, W // JW, carry=(zero,) * (4 * len(groups)),
                unroll=2)(jstep)
            for gi, g in enumerate(groups):
                a = accs[gi * 4:gi * 4 + 4]
                dot = (a[0] + a[1]) + (a[2] + a[3])
                out_v[pl.ds(cb + g * 16, 16)] = 1.0 / (1.0 + jnp.exp(-dot))

    for b in range(NBUF - 1):
        fire(b, b)

    def do_round(i, carry):
        c0 = i * NBUF
        for b in range(NBUF):
            ci = c0 + b
            drain(ci, b)
            fire(ci + NBUF - 1, (b + NBUF - 1) % NBUF)
            compute(ci, b)
        return carry

    lax.fori_loop(0, FULL_ROUNDS, do_round, 0)
    for ci in range(FULL_ROUNDS * NBUF, NCHUNK):
        b = ci % NBUF
        drain(ci, b)
        if ci + NBUF - 1 < NCHUNK:
            fire(ci + NBUF - 1, (ci + NBUF - 1) % NBUF)
        compute(ci, b)
    pltpu.sync_copy(out_v, out_hbm.at[pl.ds(base, EPW)])


def kernel(z, edge_index):
    ei = edge_index.astype(jnp.int32)
    zw = lax.bitcast_convert_type(
        z.astype(jnp.bfloat16).reshape(z.shape[0], W, 2), jnp.int32)
    return _decode(zw, ei[0], ei[1])


# speedup vs baseline: 1.5554x; 1.5554x over previous
import functools

import jax
import jax.numpy as jnp
from jax import lax
from jax.experimental import pallas as pl
from jax.experimental.pallas import tpu as pltpu
from jax.experimental.pallas import tpu_sc as plsc

E = 320000
D = 128
W = D // 2
NCORES = 2
NSUB = 16
NW = NCORES * NSUB
EPW = E // NW
C = 80
NCHUNK = EPW // C
NG = C // 16
JW = 8
NBUF = 5
FULL_ROUNDS = NCHUNK // NBUF - 1
assert C % 16 == 0 and C % 8 == 0 and EPW % C == 0 and W % JW == 0

_mesh = plsc.VectorSubcoreMesh(core_axis_name="c", subcore_axis_name="s")


@functools.partial(
    pl.kernel,
    out_type=jax.ShapeDtypeStruct((E,), jnp.float32),
    mesh=_mesh,
    scratch_types=(
        [pltpu.VMEM((EPW,), jnp.int32)] * 2
        + [pltpu.VMEM((C, W), jnp.int32)] * (2 * NBUF)
        + [pltpu.VMEM((EPW,), jnp.float32)]
        + [pltpu.SemaphoreType.DMA] * (2 * NBUF)
    ),
    compiler_params=pltpu.CompilerParams(
        needs_layout_passes=False, use_tc_tiling_on_sc=False),
)
def _decode(zw_hbm, src_hbm, dst_hbm, out_hbm, src_v, dst_v, *rest):
    rows = rest[:2 * NBUF]
    out_v = rest[2 * NBUF]
    sems = rest[2 * NBUF + 1:]
    bufs = tuple(
        (rows[2 * b], rows[2 * b + 1], sems[2 * b], sems[2 * b + 1])
        for b in range(NBUF))

    wid = lax.axis_index("s") * NCORES + lax.axis_index("c")
    base = wid * EPW
    pltpu.sync_copy(src_hbm.at[pl.ds(base, EPW)], src_v)
    pltpu.sync_copy(dst_hbm.at[pl.ds(base, EPW)], dst_v)

    lane = lax.iota(jnp.int32, 16)

    def fire(ci, b):
        rs, rd, ss, sd = bufs[b]
        cb = ci * C
        pltpu.async_copy(zw_hbm.at[src_v.at[pl.ds(cb, C)]], rs, ss)
        pltpu.async_copy(zw_hbm.at[dst_v.at[pl.ds(cb, C)]], rd, sd)

    def drain(ci, b):
        rs, rd, ss, sd = bufs[b]
        cb = ci * C
        pltpu.make_async_copy(
            zw_hbm.at[src_v.at[pl.ds(cb, C)]], rs, ss).wait()
        pltpu.make_async_copy(
            zw_hbm.at[dst_v.at[pl.ds(cb, C)]], rd, sd).wait()

    def compute(ci, b):
        rows_s, rows_d, _, _ = bufs[b]
        cb = ci * C

        def do_group(g):
            ridx = lane + g * 16
            zero = jnp.zeros((16,), jnp.float32)

            def jstep(jc, accs):
                a0, a1, a2, a3 = accs
                jb = jc * JW
                parts = []
                for k in range(0, JW, 2):
                    cw0 = (lane + (jb + k)) & (W - 1)
                    cw1 = (lane + (jb + k + 1)) & (W - 1)
                    p0 = (plsc.bitcast(plsc.load_gather(rows_s, [ridx, cw0]),
                                       jnp.bfloat16)
                          * plsc.bitcast(plsc.load_gather(rows_d, [ridx, cw0]),
                                         jnp.bfloat16))
                    p1 = (plsc.bitcast(plsc.load_gather(rows_s, [ridx, cw1]),
                                       jnp.bfloat16)
                          * plsc.bitcast(plsc.load_gather(rows_d, [ridx, cw1]),
                                         jnp.bfloat16))
                    parts.append(plsc.unpack(
                        p0 + p1, format=plsc.PackFormat.INTERLEAVED))
                for k in range(0, JW // 2, 2):
                    lo0, hi0 = parts[k]
                    lo1, hi1 = parts[k + 1]
                    a0 = a0 + lo0
                    a1 = a1 + hi0
                    a2 = a2 + lo1
                    a3 = a3 + hi1
                return (a0, a1, a2, a3)

            a0, a1, a2, a3 = plsc.parallel_loop(
                0, W // JW, carry=(zero, zero, zero, zero), unroll=2)(jstep)
            dot = (a0 + a1) + (a2 + a3)
            out_v[pl.ds(cb + g * 16, 16)] = 1.0 / (1.0 + jnp.exp(-dot))

        plsc.parallel_loop(0, NG)(do_group)

    for b in range(NBUF - 1):
        fire(b, b)

    def do_round(i, carry):
        c0 = i * NBUF
        for b in range(NBUF):
            ci = c0 + b
            drain(ci, b)
            fire(ci + NBUF - 1, (b + NBUF - 1) % NBUF)
            compute(ci, b)
        return carry

    lax.fori_loop(0, FULL_ROUNDS, do_round, 0)
    for ci in range(FULL_ROUNDS * NBUF, NCHUNK):
        b = ci % NBUF
        drain(ci, b)
        if ci + NBUF - 1 < NCHUNK:
            fire(ci + NBUF - 1, (ci + NBUF - 1) % NBUF)
        compute(ci, b)
    pltpu.sync_copy(out_v, out_hbm.at[pl.ds(base, EPW)])


def kernel(z, edge_index):
    ei = edge_index.astype(jnp.int32)
    zw = lax.bitcast_convert_type(
        z.astype(jnp.bfloat16).reshape(z.shape[0], W, 2), jnp.int32)
    return _decode(zw, ei[0], ei[1])
